# baseline (speedup 1.0000x reference)
import functools

import jax
import jax.numpy as jnp
from jax import lax
from jax.experimental import pallas as pl
from jax.experimental.pallas import tpu as pltpu
from jax.experimental.pallas import tpu_sc as plsc

N_NODES = 50000
D = 256
DEG = 16
B = 16384
B2 = 2 * B

NC = 2
NS = 16
NW = NC * NS
NPW = B2 // NW
SB = 128
NSB = NPW // SB
K = 4
RING = 4
NG = SB // K
LANES = 16
DW = D // 2


def _tree_sum(rows):
    rows = list(rows)
    while len(rows) > 1:
        rows = [rows[i] + rows[i + 1] for i in range(0, len(rows) - 1, 2)] + (
            [rows[-1]] if len(rows) % 2 else [])
    return rows[0]


def _sc_gather(nodes, neigh, feat_pk):
    mesh = plsc.VectorSubcoreMesh(core_axis_name="c", subcore_axis_name="s")

    @functools.partial(
        pl.kernel,
        mesh=mesh,
        out_type=[
            jax.ShapeDtypeStruct((B2, DW), jnp.int32),
            jax.ShapeDtypeStruct((B2, DW), jnp.int32),
        ],
        scratch_types=[
            pltpu.VMEM((NPW,), jnp.int32),
            pltpu.VMEM((SB, 128), jnp.int32),
            pltpu.VMEM((SB, DW), jnp.int32),
            pltpu.VMEM((SB, DW), jnp.int32),
            [pltpu.VMEM((K * DEG, DW), jnp.int32)] * RING,
            pltpu.SemaphoreType.DMA,
            pltpu.SemaphoreType.DMA,
            [pltpu.SemaphoreType.DMA] * RING,
        ],
    )
    def sc_kernel(nodes_hbm, neigh_hbm, feat_hbm, self_out, agg_out,
                  ids_v, nid2, self_buf, agg_buf, nbufs,
                  sem_nid, sem_self, sems):
        wid = lax.axis_index("s") * NC + lax.axis_index("c")
        base = wid * NPW
        pltpu.sync_copy(nodes_hbm.at[pl.ds(base, NPW)], ids_v)

        def fire(g, r):
            for i in range(K):
                pltpu.async_copy(
                    feat_hbm.at[nid2.at[g * K + i, pl.ds(0, DEG)]],
                    nbufs[r].at[pl.ds(i * DEG, DEG)],
                    sems[r])

        def drain(r):
            pltpu.make_async_copy(
                feat_hbm.at[pl.ds(0, K * DEG)], nbufs[r], sems[r]).wait()

        himask = jnp.full((LANES,), -65536, jnp.int32)

        def reduce_group(g, r):
            buf = nbufs[r]
            for i in range(K):
                for c in range(DW // LANES):
                    hi_acc = None
                    lo_acc = None
                    for row in range(DEG):
                        w = buf[i * DEG + row, pl.ds(c * LANES, LANES)]
                        hi = lax.bitcast_convert_type(w & himask, jnp.float32)
                        lo = lax.bitcast_convert_type(w << 16, jnp.float32)
                        hi_acc = hi if hi_acc is None else hi_acc + hi
                        lo_acc = lo if lo_acc is None else lo_acc + lo
                    hb = lax.bitcast_convert_type(hi_acc, jnp.int32) & himask
                    lb = lax.shift_right_logical(
                        lax.bitcast_convert_type(lo_acc, jnp.int32), 16)
                    agg_buf[g * K + i, pl.ds(c * LANES, LANES)] = hb | lb

        @pl.loop(0, NSB)
        def sb_loop(sb):
            nb0 = sb * SB
            idx_slice = ids_v.at[pl.ds(nb0, SB)]
            cp_nid = pltpu.async_copy(neigh_hbm.at[idx_slice], nid2, sem_nid)
            cp_self = pltpu.async_copy(feat_hbm.at[idx_slice], self_buf,
                                       sem_self)
            cp_nid.wait()
            for r in range(RING - 1):
                fire(r, r)

            @pl.loop(0, NG, step=RING)
            def g_loop(g):
                for r in range(RING):
                    @pl.when(g + r + RING - 1 < NG)
                    def _():
                        fire(g + r + RING - 1, (r + RING - 1) % RING)

                    drain(r)
                    reduce_group(g + r, r)

            cp_self.wait()
            pltpu.sync_copy(self_buf, self_out.at[pl.ds(base + nb0, SB)])
            pltpu.sync_copy(agg_buf, agg_out.at[pl.ds(base + nb0, SB)])

    return sc_kernel(nodes, neigh, feat_pk)


def _tc_score(self_all, agg_all, wut, wub, wvt, wvb):
    BLK = 2048
    nbv = B // BLK

    def body(su, au, sv, av, w_ut, w_ub, w_vt, w_vb, out):
        hu = jnp.maximum(
            jnp.dot(su[...], w_ut[...], preferred_element_type=jnp.float32)
            + jnp.dot(au[...], w_ub[...], preferred_element_type=jnp.float32),
            0.0)
        hv = jnp.maximum(
            jnp.dot(sv[...], w_vt[...], preferred_element_type=jnp.float32)
            + jnp.dot(av[...], w_vb[...], preferred_element_type=jnp.float32),
            0.0)
        out[...] = jnp.sum(hu * hv, axis=1)

    return pl.pallas_call(
        body,
        grid=(B // BLK,),
        in_specs=[
            pl.BlockSpec((BLK, D), lambda i: (i, 0)),
            pl.BlockSpec((BLK, D), lambda i: (i, 0)),
            pl.BlockSpec((BLK, D), lambda i: (i + nbv, 0)),
            pl.BlockSpec((BLK, D), lambda i: (i + nbv, 0)),
            pl.BlockSpec((D, D), lambda i: (0, 0)),
            pl.BlockSpec((D, D), lambda i: (0, 0)),
            pl.BlockSpec((D, D), lambda i: (0, 0)),
            pl.BlockSpec((D, D), lambda i: (0, 0)),
        ],
        out_specs=pl.BlockSpec((BLK,), lambda i: (i,)),
        out_shape=jax.ShapeDtypeStruct((B,), jnp.float32),
    )(self_all, agg_all, self_all, agg_all, wut, wub, wvt, wvb)


def kernel(nodes_u, nodes_v, feat, neigh_idx, W_u, W_v):
    nodes = jnp.concatenate(
        [nodes_u.astype(jnp.int32), nodes_v.astype(jnp.int32)])
    neigh_pad = jnp.pad(neigh_idx.astype(jnp.int32), ((0, 0), (0, 112)))
    feat_pk = jax.lax.bitcast_convert_type(
        feat.astype(jnp.bfloat16).reshape(N_NODES, DW, 2), jnp.int32)
    self_pk, agg_pk = _sc_gather(nodes, neigh_pad, feat_pk)
    self_o = jax.lax.bitcast_convert_type(
        self_pk, jnp.bfloat16).reshape(B2, D)
    agg_o = jax.lax.bitcast_convert_type(
        agg_pk, jnp.bfloat16).reshape(B2, D)
    wut = W_u[:D].astype(jnp.bfloat16)
    wub = (W_u[D:] * (1.0 / DEG)).astype(jnp.bfloat16)
    wvt = W_v[:D].astype(jnp.bfloat16)
    wvb = (W_v[D:] * (1.0 / DEG)).astype(jnp.bfloat16)
    return _tc_score(self_o, agg_o, wut, wub, wvt, wvb)


# speedup vs baseline: 1.0044x; 1.0044x over previous
import functools

import jax
import jax.numpy as jnp
from jax import lax
from jax.experimental import pallas as pl
from jax.experimental.pallas import tpu as pltpu
from jax.experimental.pallas import tpu_sc as plsc

N_NODES = 50000
D = 256
DEG = 16
B = 16384
B2 = 2 * B

NC = 2
NS = 16
NW = NC * NS
NPW = B2 // NW
SB = 128
NSB = NPW // SB
K = 4
RING = 4
NG = SB // K
LANES = 16
DW = D // 2


def _tree_sum(rows):
    rows = list(rows)
    while len(rows) > 1:
        rows = [rows[i] + rows[i + 1] for i in range(0, len(rows) - 1, 2)] + (
            [rows[-1]] if len(rows) % 2 else [])
    return rows[0]


def _sc_gather(nodes, neigh, feat_pk):
    mesh = plsc.VectorSubcoreMesh(core_axis_name="c", subcore_axis_name="s")

    @functools.partial(
        pl.kernel,
        mesh=mesh,
        out_type=[
            jax.ShapeDtypeStruct((B2, DW), jnp.int32),
            jax.ShapeDtypeStruct((B2, DW), jnp.int32),
        ],
        scratch_types=[
            pltpu.VMEM((NPW,), jnp.int32),
            pltpu.VMEM((SB, 128), jnp.int32),
            pltpu.VMEM((SB, DW), jnp.int32),
            pltpu.VMEM((SB, DW), jnp.int32),
            [pltpu.VMEM((K * DEG, DW), jnp.int32)] * RING,
            pltpu.SemaphoreType.DMA,
            pltpu.SemaphoreType.DMA,
            [pltpu.SemaphoreType.DMA] * RING,
        ],
    )
    def sc_kernel(nodes_hbm, neigh_hbm, feat_hbm, self_out, agg_out,
                  ids_v, nid2, self_buf, agg_buf, nbufs,
                  sem_nid, sem_self, sems):
        wid = lax.axis_index("s") * NC + lax.axis_index("c")
        base = wid * NPW
        pltpu.sync_copy(nodes_hbm.at[pl.ds(base, NPW)], ids_v)

        def fire(g, r):
            for i in range(K):
                pltpu.async_copy(
                    feat_hbm.at[nid2.at[g * K + i, pl.ds(0, DEG)]],
                    nbufs[r].at[pl.ds(i * DEG, DEG)],
                    sems[r])

        def drain(r):
            pltpu.make_async_copy(
                feat_hbm.at[pl.ds(0, K * DEG)], nbufs[r], sems[r]).wait()

        himask = jnp.full((LANES,), -65536, jnp.int32)

        def reduce_group(g, r, agg_buf):
            buf = nbufs[r]
            for i in range(K):
                for c in range(DW // LANES):
                    hi_acc = None
                    lo_acc = None
                    for row in range(DEG):
                        w = buf[i * DEG + row, pl.ds(c * LANES, LANES)]
                        hi = lax.bitcast_convert_type(w & himask, jnp.float32)
                        lo = lax.bitcast_convert_type(w << 16, jnp.float32)
                        hi_acc = hi if hi_acc is None else hi_acc + hi
                        lo_acc = lo if lo_acc is None else lo_acc + lo
                    hb = lax.bitcast_convert_type(hi_acc, jnp.int32) & himask
                    lb = lax.shift_right_logical(
                        lax.bitcast_convert_type(lo_acc, jnp.int32), 16)
                    agg_buf[g * K + i, pl.ds(c * LANES, LANES)] = hb | lb

        @pl.loop(0, NSB)
        def sb_loop(sb):
            nb0 = sb * SB
            idx_slice = ids_v.at[pl.ds(nb0, SB)]
            cp_nid = pltpu.async_copy(neigh_hbm.at[idx_slice], nid2, sem_nid)
            cp_self = pltpu.async_copy(feat_hbm.at[idx_slice], self_buf,
                                       sem_self)
            cp_nid.wait()
            for r in range(RING - 1):
                fire(r, r)

            @pl.loop(0, NG, step=RING)
            def g_loop(g):
                for r in range(RING):
                    @pl.when(g + r + RING - 1 < NG)
                    def _():
                        fire(g + r + RING - 1, (r + RING - 1) % RING)

                    drain(r)
                    reduce_group(g + r, r, agg_buf)

            cp_self.wait()
            pltpu.sync_copy(self_buf, self_out.at[pl.ds(base + nb0, SB)])
            pltpu.sync_copy(agg_buf, agg_out.at[pl.ds(base + nb0, SB)])

    return sc_kernel(nodes, neigh, feat_pk)


def _tc_score(self_all, agg_all, wut, wub, wvt, wvb):
    BLK = 2048
    nbv = B // BLK

    def body(su, au, sv, av, w_ut, w_ub, w_vt, w_vb, out):
        hu = jnp.maximum(
            jnp.dot(su[...], w_ut[...], preferred_element_type=jnp.float32)
            + jnp.dot(au[...], w_ub[...], preferred_element_type=jnp.float32),
            0.0)
        hv = jnp.maximum(
            jnp.dot(sv[...], w_vt[...], preferred_element_type=jnp.float32)
            + jnp.dot(av[...], w_vb[...], preferred_element_type=jnp.float32),
            0.0)
        out[...] = jnp.sum(hu * hv, axis=1)

    return pl.pallas_call(
        body,
        grid=(B // BLK,),
        in_specs=[
            pl.BlockSpec((BLK, D), lambda i: (i, 0)),
            pl.BlockSpec((BLK, D), lambda i: (i, 0)),
            pl.BlockSpec((BLK, D), lambda i: (i + nbv, 0)),
            pl.BlockSpec((BLK, D), lambda i: (i + nbv, 0)),
            pl.BlockSpec((D, D), lambda i: (0, 0)),
            pl.BlockSpec((D, D), lambda i: (0, 0)),
            pl.BlockSpec((D, D), lambda i: (0, 0)),
            pl.BlockSpec((D, D), lambda i: (0, 0)),
        ],
        out_specs=pl.BlockSpec((BLK,), lambda i: (i,)),
        out_shape=jax.ShapeDtypeStruct((B,), jnp.float32),
    )(self_all, agg_all, self_all, agg_all, wut, wub, wvt, wvb)


def kernel(nodes_u, nodes_v, feat, neigh_idx, W_u, W_v):
    nodes = jnp.concatenate(
        [nodes_u.astype(jnp.int32), nodes_v.astype(jnp.int32)])
    neigh_pad = jnp.pad(neigh_idx.astype(jnp.int32), ((0, 0), (0, 112)))
    feat_pk = jax.lax.bitcast_convert_type(
        feat.astype(jnp.bfloat16).reshape(N_NODES, DW, 2), jnp.int32)
    self_pk, agg_pk = _sc_gather(nodes, neigh_pad, feat_pk)
    self_o = jax.lax.bitcast_convert_type(
        self_pk, jnp.bfloat16).reshape(B2, D)
    agg_o = jax.lax.bitcast_convert_type(
        agg_pk, jnp.bfloat16).reshape(B2, D)
    wut = W_u[:D].astype(jnp.bfloat16)
    wub = (W_u[D:] * (1.0 / DEG)).astype(jnp.bfloat16)
    wvt = W_v[:D].astype(jnp.bfloat16)
    wvb = (W_v[D:] * (1.0 / DEG)).astype(jnp.bfloat16)
    return _tc_score(self_o, agg_o, wut, wub, wvt, wvb)
